# baseline (speedup 1.0000x reference)
import functools

import jax
import jax.numpy as jnp
from jax import lax
from jax.experimental import pallas as pl
from jax.experimental.pallas import tpu as pltpu
from jax.experimental.pallas import tpu_sc as plsc

_NUM_WORKERS = 32
_CHUNK = 128


@functools.partial(jax.jit, static_argnames=())
def _gather2(batch_e1, batch_q, emb_e, emb_r):
    B = batch_e1.shape[0]
    D = emb_e.shape[1]
    b_per_w = B // _NUM_WORKERS
    n_chunks = b_per_w // _CHUNK

    mesh = plsc.VectorSubcoreMesh(core_axis_name="c", subcore_axis_name="s")

    @functools.partial(
        pl.kernel,
        mesh=mesh,
        out_type=(
            jax.ShapeDtypeStruct((B, D), jnp.float32),
            jax.ShapeDtypeStruct((B, D), jnp.float32),
        ),
        scratch_types=[
            pltpu.VMEM((b_per_w,), jnp.int32),
            pltpu.VMEM((b_per_w,), jnp.int32),
            pltpu.VMEM((b_per_w, D), jnp.float32),
            pltpu.VMEM((b_per_w, D), jnp.float32),
            pltpu.SemaphoreType.DMA,
            pltpu.SemaphoreType.DMA,
        ],
        compiler_params=pltpu.CompilerParams(use_tc_tiling_on_sc=False),
    )
    def k(e1_hbm, q_hbm, emb_e_hbm, emb_r_hbm, out_h_hbm, out_q_hbm,
          idx_e, idx_q, rows_e, rows_q, sem_e, sem_q):
        wid = lax.axis_index("s") * 2 + lax.axis_index("c")
        base = wid * b_per_w
        pltpu.sync_copy(e1_hbm.at[pl.ds(base, b_per_w)], idx_e)
        pltpu.sync_copy(q_hbm.at[pl.ds(base, b_per_w)], idx_q)
        copies = []
        for j in range(n_chunks):
            s = pl.ds(j * _CHUNK, _CHUNK)
            copies.append(
                pltpu.async_copy(emb_e_hbm.at[idx_e.at[s]], rows_e.at[s], sem_e))
            copies.append(
                pltpu.async_copy(emb_r_hbm.at[idx_q.at[s]], rows_q.at[s], sem_q))
        for cp in copies:
            cp.wait()
        pltpu.sync_copy(rows_e, out_h_hbm.at[pl.ds(base, b_per_w)])
        pltpu.sync_copy(rows_q, out_q_hbm.at[pl.ds(base, b_per_w)])

    return k(batch_e1, batch_q, emb_e, emb_r)


def kernel(batch_e1, batch_q, emb_e, emb_r):
    return _gather2(batch_e1.astype(jnp.int32), batch_q.astype(jnp.int32),
                    emb_e, emb_r)
